# baseline (speedup 1.0000x reference)
import functools

import numpy as np
import jax
import jax.numpy as jnp
from jax import lax
from jax.experimental import pallas as pl
from jax.experimental.pallas import tpu as pltpu
from jax.experimental.pallas import tpu_sc as plsc

N_NODES = 10000
N_EDGES = 320000
FIN = 4
FOUT = 4
NB = 8
MB = 8
NTILES = 32
CHUNK = 512
NG = 4
RS_PAD = 10008
ACC_W = N_NODES * FOUT


def _tec_body(pos_h, feat_h, tf_h, rs_h, w_h, bounds_h, out_h,
              pos_v, feat_v, rs_v, nb_v, acc, w_v, b_v, cs_v, w_s):
    wid = lax.axis_index("s") * 2 + lax.axis_index("c")
    pltpu.sync_copy(bounds_h.at[pl.ds(pl.multiple_of(wid * 16, 16), 16)], b_v)
    pltpu.sync_copy(pos_h, pos_v)
    pltpu.sync_copy(feat_h, feat_v)
    pltpu.sync_copy(rs_h, rs_v)
    pltpu.sync_copy(w_h, w_v)

    bvec = b_v[...]
    node_lo = bvec[0]
    node_hi = bvec[1]
    edge_lo = bvec[2]
    edge_hi = bvec[3]

    zeros16f = jnp.zeros((16,), jnp.float32)
    z16 = jnp.zeros((16,), jnp.int32)
    o16 = jnp.ones((16,), jnp.int32)

    def zbody(i, c):
        acc[pl.ds(i * 16, 16)] = zeros16f
        return c

    lax.fori_loop(0, ACC_W // 16, zbody, 0)

    for k in range(FOUT * FIN * NB * MB // 16):
        wvec = w_v[pl.ds(k * 16, 16)]
        for l in range(16):
            w_s[k * 16 + l] = wvec[l]

    e_align = (edge_lo >> 6) << 6
    n_chunks = (edge_hi - e_align + (CHUNK - 1)) // CHUNK

    def chunk_body(ci, c):
        cbase = pl.multiple_of(e_align + ci * CHUNK, 64)
        pltpu.sync_copy(tf_h.at[pl.ds(cbase, CHUNK)], nb_v)
        ng = (jnp.minimum(edge_hi - cbase, CHUNK) + 63) // 64

        def group_body(gi, cc):
            valid2, segs42, h2, fs2 = [], [], [], []
            for g in range(NG):
                off = gi * 64 + g * 16
                ids = cbase + off + lax.iota(jnp.int32, 16)
                valid = (ids >= edge_lo) & (ids < edge_hi)
                nbv = nb_v[pl.ds(off, 16)]
                nb2 = nbv << 1
                nb4 = nbv << 2

                lo = jnp.full((16,), node_lo, jnp.int32)
                hi = jnp.full((16,), node_hi, jnp.int32)
                for _ in range(14):
                    mid = (lo + hi) >> 1
                    v = plsc.load_gather(rs_v, [mid])
                    p = v <= ids
                    lo = jnp.where(p, mid, lo)
                    hi = jnp.where(p, hi, mid)
                seg = lo

                seg2 = seg << 1
                pxs = plsc.load_gather(pos_v, [nb2])
                pys = plsc.load_gather(pos_v, [nb2 + 1])
                pxd = plsc.load_gather(pos_v, [seg2])
                pyd = plsc.load_gather(pos_v, [seg2 + 1])
                h = [plsc.load_gather(feat_v, [nb4 + j]) for j in range(FIN)]

                dx = pxs - pxd
                dy = pys - pyd
                s = dx * dx + dy * dy
                bi = jnp.int32(0x5F3759DF) - (plsc.bitcast(s, jnp.int32) >> 1)
                y = plsc.bitcast(bi, jnp.float32)
                for _ in range(3):
                    y = y * (1.5 - 0.5 * s * y * y)
                nz = s > 0.0
                inv = jnp.where(nz, y, 0.0)
                rho = s * inv
                ct = jnp.where(nz, dx * inv, 1.0)
                st = dy * inv

                x = rho - 1.0
                x2 = x + x
                cs = [None] * NB
                cs[1] = x
                cs[2] = x2 * x - 1.0
                for k in range(3, NB):
                    cs[k] = x2 * cs[k - 1] - cs[k - 2]

                c2 = ct * ct - st * st
                s2 = st * ct + st * ct
                c3 = c2 * ct - s2 * st
                s3 = s2 * ct + c2 * st
                c4 = c3 * ct - s3 * st
                s4 = s3 * ct + c3 * st
                fs = [None, st, c2, s2, c3, s3, c4, s4]

                cs_v[pl.ds(g * NB * 16, 16)] = jnp.ones((16,), jnp.float32)
                for k in range(1, NB):
                    cs_v[pl.ds((g * NB + k) * 16, 16)] = cs[k]

                valid2.append(valid)
                segs42.append(seg << 2)
                h2.append(h)
                fs2.append(fs)

            zf = jnp.zeros((16,), jnp.float32)
            for i in range(FOUT):
                def nbody(n, acc_i, i=i):
                    base = (i * NB + n) * (FIN * MB)
                    t = [[] for _ in range(NG)]
                    for j in range(FIN):
                        w0 = w_s[base + j * MB]
                        ws = [w_s[base + j * MB + m] for m in range(1, MB)]
                        for g in range(NG):
                            f = fs2[g]
                            p = [ws[m - 1] * f[m] for m in range(1, MB)]
                            q0 = p[0] + p[1]
                            q1 = p[2] + p[3]
                            q2 = p[4] + p[5]
                            q3 = p[6] + w0
                            sg = (q0 + q1) + (q2 + q3)
                            t[g].append(sg * h2[g][j])
                    out = []
                    for g in range(NG):
                        tg = (t[g][0] + t[g][1]) + (t[g][2] + t[g][3])
                        cn = cs_v[pl.ds(
                            pl.multiple_of((g * NB + n) * 16, 16), 16)]
                        out.append(acc_i[g] + tg * cn)
                    return tuple(out)

                acc_i = lax.fori_loop(0, NB, nbody, (zf,) * NG)
                for g in range(NG):
                    plsc.addupdate_scatter(acc, [segs42[g] + i], acc_i[g],
                                           mask=valid2[g])
            return cc

        lax.fori_loop(0, ng, group_body, c)
        return c

    lax.fori_loop(0, n_chunks, chunk_body, 0)

    wbase = node_lo * FOUT

    def obody(k, c):
        woff = pl.multiple_of(wbase + k * 16, 16)
        pltpu.sync_copy(acc.at[pl.ds(woff, 16)], out_h.at[pl.ds(woff, 16)])
        return c

    lax.fori_loop(0, (node_hi - node_lo) * FOUT // 16, obody, 0)


@jax.jit
def _run(positions, features, tf_pad, rs_pad, w_flat, bounds):
    mesh = plsc.VectorSubcoreMesh(core_axis_name="c", subcore_axis_name="s")
    f = functools.partial(
        pl.kernel,
        mesh=mesh,
        compiler_params=pltpu.CompilerParams(needs_layout_passes=False),
        out_type=jax.ShapeDtypeStruct((ACC_W,), jnp.float32),
        scratch_types=[
            pltpu.VMEM((N_NODES * 2,), jnp.float32),
            pltpu.VMEM((N_NODES * FIN,), jnp.float32),
            pltpu.VMEM((RS_PAD,), jnp.int32),
            pltpu.VMEM((CHUNK,), jnp.int32),
            pltpu.VMEM((ACC_W,), jnp.float32),
            pltpu.VMEM((FOUT * FIN * NB * MB,), jnp.float32),
            pltpu.VMEM((16,), jnp.int32),
            pltpu.VMEM((NG * NB * 16,), jnp.float32),
            pltpu.SMEM((FOUT * FIN * NB * MB,), jnp.float32),
        ],
    )(_tec_body)
    return f(positions, features, tf_pad, rs_pad, w_flat, bounds)


def kernel(positions, features, tf_neighbors, row_splits, kernel):
    rs = row_splits.astype(jnp.int32)

    fconst = jnp.concatenate([
        jnp.full((1,), 1.0 / np.sqrt(2.0 * np.pi), jnp.float32),
        jnp.full((MB - 1,), 1.0 / np.sqrt(np.pi), jnp.float32),
    ])
    wf = kernel.astype(jnp.float32) * fconst[None, None, None, :]
    w_flat = jnp.transpose(wf, (0, 2, 1, 3)).reshape(-1)

    targets = jnp.arange(NTILES + 1, dtype=jnp.int32) * (N_EDGES // NTILES)
    node_b = jnp.clip(jnp.searchsorted(rs, targets, side="left"),
                      0, N_NODES).astype(jnp.int32)
    node_b = (node_b >> 2) << 2
    node_b = node_b.at[0].set(0).at[NTILES].set(N_NODES)
    edge_b = jnp.take(rs, node_b)
    bounds = jnp.stack([node_b[:-1], node_b[1:], edge_b[:-1], edge_b[1:]],
                       axis=1)
    bounds = jnp.pad(bounds, ((0, 0), (0, 12))).reshape(-1)

    tf_pad = jnp.concatenate(
        [tf_neighbors.astype(jnp.int32),
         jnp.zeros((CHUNK,), jnp.int32)])
    rs_pad = jnp.concatenate(
        [rs, jnp.full((RS_PAD - N_NODES - 1,), N_EDGES, jnp.int32)])

    out = _run(positions.astype(jnp.float32).reshape(-1),
               features.astype(jnp.float32).reshape(-1),
               tf_pad, rs_pad, w_flat, bounds)
    return out.reshape(N_NODES, FOUT)


# speedup vs baseline: 1.7786x; 1.7786x over previous
import functools

import numpy as np
import jax
import jax.numpy as jnp
from jax import lax
from jax.experimental import pallas as pl
from jax.experimental.pallas import tpu as pltpu
from jax.experimental.pallas import tpu_sc as plsc

N_NODES = 10000
N_EDGES = 320000
FIN = 4
FOUT = 4
NB = 8
MB = 8
NTILES = 32
CHUNK = 512
RS_PAD = 10008
ACC_W = N_NODES * FOUT


def _tec_body(pos_h, feat_h, tf_h, rs_h, w_h, bounds_h, out_h,
              pos_v, feat_v, rs_v, nb_v, acc, w_v, b_v, cs_v, w_s):
    wid = lax.axis_index("s") * 2 + lax.axis_index("c")
    pltpu.sync_copy(bounds_h.at[pl.ds(pl.multiple_of(wid * 16, 16), 16)], b_v)
    pltpu.sync_copy(pos_h, pos_v)
    pltpu.sync_copy(feat_h, feat_v)
    pltpu.sync_copy(rs_h, rs_v)
    pltpu.sync_copy(w_h, w_v)

    bvec = b_v[...]
    node_lo = bvec[0]
    node_hi = bvec[1]
    edge_lo = bvec[2]
    edge_hi = bvec[3]

    wwin = jnp.maximum(jnp.full((16,), node_hi - node_lo, jnp.int32), 1)
    wexp = (plsc.bitcast(wwin.astype(jnp.float32), jnp.int32) >> 23) - 127
    n_search = wexp[0] + 1

    zeros16f = jnp.zeros((16,), jnp.float32)
    z16 = jnp.zeros((16,), jnp.int32)
    o16 = jnp.ones((16,), jnp.int32)

    def zbody(i, c):
        acc[pl.ds(i * 16, 16)] = zeros16f
        return c

    lax.fori_loop(0, ACC_W // 16, zbody, 0)

    for k in range(FOUT * FIN * NB * MB // 16):
        wvec = w_v[pl.ds(k * 16, 16)]
        for l in range(16):
            w_s[k * 16 + l] = wvec[l]

    e_align = (edge_lo >> 5) << 5
    n_chunks = (edge_hi - e_align + (CHUNK - 1)) // CHUNK

    def chunk_body(ci, c):
        cbase = pl.multiple_of(e_align + ci * CHUNK, 32)
        pltpu.sync_copy(tf_h.at[pl.ds(cbase, CHUNK)], nb_v)
        ng = (jnp.minimum(edge_hi - cbase, CHUNK) + 31) // 32

        def group_body(gi, cc):
            R = (0, 1)
            off = [gi * 32 + g * 16 for g in R]
            ids = [cbase + off[g] + lax.iota(jnp.int32, 16) for g in R]
            valid2 = [(ids[g] >= edge_lo) & (ids[g] < edge_hi) for g in R]
            nbv = [nb_v[pl.ds(off[g], 16)] for g in R]
            nb2 = [nbv[g] << 1 for g in R]
            nb4 = [nbv[g] << 2 for g in R]

            lo0 = jnp.full((16,), node_lo, jnp.int32)
            hi0 = jnp.full((16,), node_hi, jnp.int32)

            def sbody(_, carry):
                lo, hi = carry
                mid = [(lo[g] + hi[g]) >> 1 for g in R]
                v = [plsc.load_gather(rs_v, [mid[g]]) for g in R]
                p = [v[g] <= ids[g] for g in R]
                return (tuple(jnp.where(p[g], mid[g], lo[g]) for g in R),
                        tuple(jnp.where(p[g], hi[g], mid[g]) for g in R))

            seg, _ = lax.fori_loop(0, n_search, sbody,
                                   ((lo0, lo0), (hi0, hi0)))
            seg2 = [seg[g] << 1 for g in R]
            segs42 = [seg[g] << 2 for g in R]

            pxs = [plsc.load_gather(pos_v, [nb2[g]]) for g in R]
            pys = [plsc.load_gather(pos_v, [nb2[g] + 1]) for g in R]
            pxd = [plsc.load_gather(pos_v, [seg2[g]]) for g in R]
            pyd = [plsc.load_gather(pos_v, [seg2[g] + 1]) for g in R]
            h2 = [[plsc.load_gather(feat_v, [nb4[g] + j]) for j in range(FIN)]
                  for g in R]

            dx = [pxs[g] - pxd[g] for g in R]
            dy = [pys[g] - pyd[g] for g in R]
            s = [dx[g] * dx[g] + dy[g] * dy[g] for g in R]
            y = [plsc.bitcast(
                jnp.int32(0x5F3759DF) - (plsc.bitcast(s[g], jnp.int32) >> 1),
                jnp.float32) for g in R]
            for _ in range(3):
                y = [y[g] * (1.5 - 0.5 * s[g] * y[g] * y[g]) for g in R]
            nz = [s[g] > 0.0 for g in R]
            inv = [jnp.where(nz[g], y[g], 0.0) for g in R]
            rho = [s[g] * inv[g] for g in R]
            ct = [jnp.where(nz[g], dx[g] * inv[g], 1.0) for g in R]
            st = [dy[g] * inv[g] for g in R]

            x = [rho[g] - 1.0 for g in R]
            x2 = [x[g] + x[g] for g in R]
            cs = [[None] * NB for g in R]
            for g in R:
                cs[g][1] = x[g]
                cs[g][2] = x2[g] * x[g] - 1.0
            for k in range(3, NB):
                for g in R:
                    cs[g][k] = x2[g] * cs[g][k - 1] - cs[g][k - 2]

            c2 = [ct[g] * ct[g] - st[g] * st[g] for g in R]
            s2 = [st[g] * ct[g] + st[g] * ct[g] for g in R]
            c3 = [c2[g] * ct[g] - s2[g] * st[g] for g in R]
            s3 = [s2[g] * ct[g] + c2[g] * st[g] for g in R]
            c4 = [c3[g] * ct[g] - s3[g] * st[g] for g in R]
            s4 = [s3[g] * ct[g] + c3[g] * st[g] for g in R]
            fs2 = [[None, st[g], c2[g], s2[g], c3[g], s3[g], c4[g], s4[g]]
                   for g in R]

            one16 = jnp.ones((16,), jnp.float32)
            for g in R:
                cs_v[pl.ds(g * NB * 16, 16)] = one16
                for k in range(1, NB):
                    cs_v[pl.ds((g * NB + k) * 16, 16)] = cs[g][k]

            zf = jnp.zeros((16,), jnp.float32)
            for i in range(FOUT):
                def nbody(n, acc_i, i=i):
                    base = (i * NB + n) * (FIN * MB)
                    t = [[], []]
                    for j in range(FIN):
                        w0 = w_s[base + j * MB]
                        ws = [w_s[base + j * MB + m] for m in range(1, MB)]
                        for g in range(2):
                            f = fs2[g]
                            p = [ws[m - 1] * f[m] for m in range(1, MB)]
                            q0 = p[0] + p[1]
                            q1 = p[2] + p[3]
                            q2 = p[4] + p[5]
                            q3 = p[6] + w0
                            sg = (q0 + q1) + (q2 + q3)
                            t[g].append(sg * h2[g][j])
                    out = []
                    for g in range(2):
                        tg = (t[g][0] + t[g][1]) + (t[g][2] + t[g][3])
                        cn = cs_v[pl.ds(
                            pl.multiple_of((g * NB + n) * 16, 16), 16)]
                        out.append(acc_i[g] + tg * cn)
                    return tuple(out)

                acc_i = lax.fori_loop(0, NB, nbody, (zf, zf))
                for g in range(2):
                    plsc.addupdate_scatter(acc, [segs42[g] + i], acc_i[g],
                                           mask=valid2[g])
            return cc

        lax.fori_loop(0, ng, group_body, c)
        return c

    lax.fori_loop(0, n_chunks, chunk_body, 0)

    wbase = node_lo * FOUT

    def obody(k, c):
        woff = pl.multiple_of(wbase + k * 16, 16)
        pltpu.sync_copy(acc.at[pl.ds(woff, 16)], out_h.at[pl.ds(woff, 16)])
        return c

    lax.fori_loop(0, (node_hi - node_lo) * FOUT // 16, obody, 0)


@jax.jit
def _run(positions, features, tf_pad, rs_pad, w_flat, bounds):
    mesh = plsc.VectorSubcoreMesh(core_axis_name="c", subcore_axis_name="s")
    f = functools.partial(
        pl.kernel,
        mesh=mesh,
        compiler_params=pltpu.CompilerParams(needs_layout_passes=False),
        out_type=jax.ShapeDtypeStruct((ACC_W,), jnp.float32),
        scratch_types=[
            pltpu.VMEM((N_NODES * 2,), jnp.float32),
            pltpu.VMEM((N_NODES * FIN,), jnp.float32),
            pltpu.VMEM((RS_PAD,), jnp.int32),
            pltpu.VMEM((CHUNK,), jnp.int32),
            pltpu.VMEM((ACC_W,), jnp.float32),
            pltpu.VMEM((FOUT * FIN * NB * MB,), jnp.float32),
            pltpu.VMEM((16,), jnp.int32),
            pltpu.VMEM((2 * NB * 16,), jnp.float32),
            pltpu.SMEM((FOUT * FIN * NB * MB,), jnp.float32),
        ],
    )(_tec_body)
    return f(positions, features, tf_pad, rs_pad, w_flat, bounds)


def kernel(positions, features, tf_neighbors, row_splits, kernel):
    rs = row_splits.astype(jnp.int32)

    fconst = jnp.concatenate([
        jnp.full((1,), 1.0 / np.sqrt(2.0 * np.pi), jnp.float32),
        jnp.full((MB - 1,), 1.0 / np.sqrt(np.pi), jnp.float32),
    ])
    wf = kernel.astype(jnp.float32) * fconst[None, None, None, :]
    w_flat = jnp.transpose(wf, (0, 2, 1, 3)).reshape(-1)

    targets = jnp.arange(NTILES + 1, dtype=jnp.int32) * (N_EDGES // NTILES)
    node_b = jnp.clip(jnp.searchsorted(rs, targets, side="left"),
                      0, N_NODES).astype(jnp.int32)
    node_b = (node_b >> 2) << 2
    node_b = node_b.at[0].set(0).at[NTILES].set(N_NODES)
    edge_b = jnp.take(rs, node_b)
    bounds = jnp.stack([node_b[:-1], node_b[1:], edge_b[:-1], edge_b[1:]],
                       axis=1)
    bounds = jnp.pad(bounds, ((0, 0), (0, 12))).reshape(-1)

    tf_pad = jnp.concatenate(
        [tf_neighbors.astype(jnp.int32),
         jnp.zeros((CHUNK,), jnp.int32)])
    rs_pad = jnp.concatenate(
        [rs, jnp.full((RS_PAD - N_NODES - 1,), N_EDGES, jnp.int32)])

    out = _run(positions.astype(jnp.float32).reshape(-1),
               features.astype(jnp.float32).reshape(-1),
               tf_pad, rs_pad, w_flat, bounds)
    return out.reshape(N_NODES, FOUT)
